# Initial kernel scaffold; baseline (speedup 1.0000x reference)
import jax
import jax.numpy as jnp
from jax import lax
from jax.experimental import pallas as pl
from jax.experimental.pallas import tpu as pltpu

B_, L_, DM = 2, 2048, 1024
DS, DC, HD = 128, 4, 64
DI = 2048
NH = DI // HD
CONV_DIM = DI + 2 * DS
CS = 256
NC = L_ // CS
EPS = 1e-5
BF = jnp.bfloat16
F32 = jnp.float32


def _silu(v):
    return v * (1.0 / (1.0 + jnp.exp(-v)))


def _softplus(v):
    return jnp.maximum(v, 0.0) + jnp.log(1.0 + jnp.exp(-jnp.abs(v)))


def _body(x_ref, nw_ref, wz_ref, wxbc_ref, wdt_ref, convw_ref, convb_ref,
          dtb_ref, alog_ref, d_ref, gw_ref, wout_ref,
          out_ref, state_ref, hist_ref, y_ref):
    c = pl.program_id(1)

    @pl.when(c == 0)
    def _():
        state_ref[...] = jnp.zeros_like(state_ref)
        hist_ref[...] = jnp.zeros_like(hist_ref)

    xb = x_ref[0]
    ms = jnp.mean(xb * xb, axis=1, keepdims=True)
    xn = xb * lax.rsqrt(ms + EPS) * nw_ref[...]
    xnb = xn.astype(BF)

    dnums = (((1,), (1,)), ((), ()))
    z = lax.dot_general(xnb, wz_ref[...], dnums, preferred_element_type=F32)
    xbc_raw = lax.dot_general(xnb, wxbc_ref[...], dnums, preferred_element_type=F32)
    dtr = lax.dot_general(xnb, wdt_ref[...], dnums, preferred_element_type=F32)

    ext = jnp.concatenate([hist_ref[...], xbc_raw], axis=0)
    cw = convw_ref[...]
    conv = (ext[5:5 + CS] * cw[0:1] + ext[6:6 + CS] * cw[1:2]
            + ext[7:7 + CS] * cw[2:3] + ext[8:8 + CS] * cw[3:4]) + convb_ref[...]
    xBC = _silu(conv)
    hist_ref[5:8, :] = xbc_raw[CS - 3:CS, :]

    xs = xBC[:, :DI]
    Bm = xBC[:, DI:DI + DS]
    Cm = xBC[:, DI + DS:]

    dt = _softplus(dtr + dtb_ref[...])
    A = -jnp.exp(alog_ref[...])
    dtA = dt * A
    Acs = jnp.cumsum(dtA, axis=0)
    AcsT = jnp.swapaxes(Acs, 0, 1)
    Asum = Acs[CS - 1:CS, :]
    gam = jnp.exp(Asum)
    dec_st = jnp.exp(Asum - Acs)
    expAcs = jnp.exp(Acs)

    Bmb = Bm.astype(BF)
    Cmb = Cm.astype(BF)
    CB = lax.dot_general(Cmb, Bmb, dnums, preferred_element_type=F32)
    li = lax.broadcasted_iota(jnp.int32, (CS, CS), 0)
    si = lax.broadcasted_iota(jnp.int32, (CS, CS), 1)
    mask = li >= si

    tA = (((0,), (0,)), ((), ()))
    for h in range(NH):
        sl = slice(h * HD, (h + 1) * HD)
        ah_col = Acs[:, h:h + 1]
        ah_row = AcsT[h:h + 1, :]
        Mh = jnp.where(mask, jnp.exp(ah_col - ah_row) * CB, 0.0)
        Xh = xs[:, sl] * dt[:, h:h + 1]
        Yd = jnp.dot(Mh.astype(BF), Xh.astype(BF), preferred_element_type=F32)
        st = state_ref[h * HD:(h + 1) * HD, :]
        Yo = lax.dot_general(Cmb, st.astype(BF), dnums, preferred_element_type=F32)
        Xw = (Xh * dec_st[:, h:h + 1]).astype(BF)
        upd = lax.dot_general(Xw, Bmb, tA, preferred_element_type=F32)
        state_ref[h * HD:(h + 1) * HD, :] = st * gam[0:1, h:h + 1] + upd
        y_ref[:, sl] = (Yd + Yo * expAcs[:, h:h + 1]
                        + d_ref[0:1, h:h + 1] * xs[:, sl])

    y = y_ref[...] * _silu(z)
    ms2 = jnp.mean(y * y, axis=1, keepdims=True)
    yn = y * lax.rsqrt(ms2 + EPS) * gw_ref[...]
    outp = lax.dot_general(yn.astype(BF), wout_ref[...], dnums,
                           preferred_element_type=F32)
    out_ref[0] = xb + outp


def kernel(x, norm_w, in_proj_w, conv_w, conv_b, dt_bias, A_log, D, gnorm_w,
           out_proj_w):
    wz = in_proj_w[:DI].astype(BF)
    wxbc = in_proj_w[DI:DI + CONV_DIM].astype(BF)
    wdt = in_proj_w[DI + CONV_DIM:].astype(BF)
    convw = jnp.zeros((8, CONV_DIM), F32).at[:DC].set(conv_w.T)

    grid = (B_, NC)
    const = lambda b, c: (0, 0)
    out = pl.pallas_call(
        _body,
        out_shape=jax.ShapeDtypeStruct((B_, L_, DM), F32),
        grid=grid,
        in_specs=[
            pl.BlockSpec((1, CS, DM), lambda b, c: (b, c, 0)),
            pl.BlockSpec((1, DM), const),
            pl.BlockSpec((DI, DM), const),
            pl.BlockSpec((CONV_DIM, DM), const),
            pl.BlockSpec((NH, DM), const),
            pl.BlockSpec((8, CONV_DIM), const),
            pl.BlockSpec((1, CONV_DIM), const),
            pl.BlockSpec((1, NH), const),
            pl.BlockSpec((1, NH), const),
            pl.BlockSpec((1, NH), const),
            pl.BlockSpec((1, DI), const),
            pl.BlockSpec((DM, DI), const),
        ],
        out_specs=pl.BlockSpec((1, CS, DM), lambda b, c: (b, c, 0)),
        scratch_shapes=[
            pltpu.VMEM((NH * HD, DS), F32),
            pltpu.VMEM((8, CONV_DIM), F32),
            pltpu.VMEM((CS, DI), F32),
        ],
        compiler_params=pltpu.CompilerParams(
            dimension_semantics=("parallel", "arbitrary"),
        ),
        name="mamba2_block",
    )(x, norm_w.reshape(1, DM), wz, wxbc, wdt, convw, conv_b.reshape(1, -1),
      dt_bias.reshape(1, -1), A_log.reshape(1, -1), D.reshape(1, -1),
      gnorm_w.reshape(1, -1), out_proj_w.astype(BF))
    return out


# speedup vs baseline: 1.5794x; 1.5794x over previous
import jax
import jax.numpy as jnp
from jax import lax
from jax.experimental import pallas as pl
from jax.experimental.pallas import tpu as pltpu

B_, L_, DM = 2, 2048, 1024
DS, DC, HD = 128, 4, 64
DI = 2048
NH = DI // HD
CONV_DIM = DI + 2 * DS
CS = 256
NC = L_ // CS
EPS = 1e-5
BF = jnp.bfloat16
F32 = jnp.float32


def _silu(v):
    return v * (1.0 / (1.0 + jnp.exp(-v)))


def _softplus(v):
    return jnp.maximum(v, 0.0) + jnp.log(1.0 + jnp.exp(-jnp.abs(v)))


def _body(x_ref, nw_ref, wz_ref, wxbc_ref, wdt_ref, convw_ref, convb_ref,
          dtb_ref, alog_ref, d_ref, gw_ref, wout_ref,
          out_ref, state_ref, hist_ref, y_ref):
    c = pl.program_id(1)

    @pl.when(c == 0)
    def _():
        state_ref[...] = jnp.zeros_like(state_ref)
        hist_ref[...] = jnp.zeros_like(hist_ref)

    xb = x_ref[0]
    ms = jnp.mean(xb * xb, axis=1, keepdims=True)
    xn = xb * lax.rsqrt(ms + EPS) * nw_ref[...]
    xnb = xn.astype(BF)

    dnums = (((1,), (1,)), ((), ()))
    z = lax.dot_general(xnb, wz_ref[...], dnums, preferred_element_type=F32)
    xbc_raw = lax.dot_general(xnb, wxbc_ref[...], dnums, preferred_element_type=F32)
    dtr = lax.dot_general(xnb, wdt_ref[...], dnums, preferred_element_type=F32)

    ext = jnp.concatenate([hist_ref[...], xbc_raw], axis=0)
    cw = convw_ref[...]
    conv = (ext[5:5 + CS] * cw[0:1] + ext[6:6 + CS] * cw[1:2]
            + ext[7:7 + CS] * cw[2:3] + ext[8:8 + CS] * cw[3:4]) + convb_ref[...]
    xBC = _silu(conv)
    hist_ref[5:8, :] = xbc_raw[CS - 3:CS, :]

    xs = xBC[:, :DI]
    Bm = xBC[:, DI:DI + DS]
    Cm = xBC[:, DI + DS:]

    li = lax.broadcasted_iota(jnp.int32, (CS, CS), 0)
    si = lax.broadcasted_iota(jnp.int32, (CS, CS), 1)
    mask = li >= si
    tril = jnp.where(mask, 1.0, 0.0).astype(BF)

    dt = _softplus(dtr + dtb_ref[...])
    A = -jnp.exp(alog_ref[...])
    dtA = dt * A
    dtA_hi = dtA.astype(BF)
    dtA_lo = (dtA - dtA_hi.astype(F32)).astype(BF)
    Acs = (jnp.dot(tril, dtA_hi, preferred_element_type=F32)
           + jnp.dot(tril, dtA_lo, preferred_element_type=F32))
    AcsT = jnp.swapaxes(Acs, 0, 1)
    Asum = Acs[CS - 1:CS, :]
    gam = jnp.exp(Asum)
    dec_st = jnp.exp(Asum - Acs)
    expAcs = jnp.exp(Acs)

    Bmb = Bm.astype(BF)
    Cmb = Cm.astype(BF)
    CB = lax.dot_general(Cmb, Bmb, dnums, preferred_element_type=F32)

    tA = (((0,), (0,)), ((), ()))
    for h in range(NH):
        sl = slice(h * HD, (h + 1) * HD)
        ah_col = Acs[:, h:h + 1]
        ah_row = AcsT[h:h + 1, :]
        Mh = jnp.where(mask, jnp.exp(ah_col - ah_row) * CB, 0.0)
        Xh = xs[:, sl] * dt[:, h:h + 1]
        Yd = jnp.dot(Mh.astype(BF), Xh.astype(BF), preferred_element_type=F32)
        st = state_ref[h * HD:(h + 1) * HD, :]
        Yo = lax.dot_general(Cmb, st.astype(BF), dnums, preferred_element_type=F32)
        Xw = (Xh * dec_st[:, h:h + 1]).astype(BF)
        upd = lax.dot_general(Xw, Bmb, tA, preferred_element_type=F32)
        state_ref[h * HD:(h + 1) * HD, :] = st * gam[0:1, h:h + 1] + upd
        y_ref[:, sl] = (Yd + Yo * expAcs[:, h:h + 1]
                        + d_ref[0:1, h:h + 1] * xs[:, sl])

    y = y_ref[...] * _silu(z)
    ms2 = jnp.mean(y * y, axis=1, keepdims=True)
    yn = y * lax.rsqrt(ms2 + EPS) * gw_ref[...]
    outp = lax.dot_general(yn.astype(BF), wout_ref[...], dnums,
                           preferred_element_type=F32)
    out_ref[0] = xb + outp


def kernel(x, norm_w, in_proj_w, conv_w, conv_b, dt_bias, A_log, D, gnorm_w,
           out_proj_w):
    wz = in_proj_w[:DI].astype(BF)
    wxbc = in_proj_w[DI:DI + CONV_DIM].astype(BF)
    wdt = in_proj_w[DI + CONV_DIM:].astype(BF)
    convw = jnp.zeros((8, CONV_DIM), F32).at[:DC].set(conv_w.T)

    grid = (B_, NC)
    const = lambda b, c: (0, 0)
    out = pl.pallas_call(
        _body,
        out_shape=jax.ShapeDtypeStruct((B_, L_, DM), F32),
        grid=grid,
        in_specs=[
            pl.BlockSpec((1, CS, DM), lambda b, c: (b, c, 0)),
            pl.BlockSpec((1, DM), const),
            pl.BlockSpec((DI, DM), const),
            pl.BlockSpec((CONV_DIM, DM), const),
            pl.BlockSpec((NH, DM), const),
            pl.BlockSpec((8, CONV_DIM), const),
            pl.BlockSpec((1, CONV_DIM), const),
            pl.BlockSpec((1, NH), const),
            pl.BlockSpec((1, NH), const),
            pl.BlockSpec((1, NH), const),
            pl.BlockSpec((1, DI), const),
            pl.BlockSpec((DM, DI), const),
        ],
        out_specs=pl.BlockSpec((1, CS, DM), lambda b, c: (b, c, 0)),
        scratch_shapes=[
            pltpu.VMEM((NH * HD, DS), F32),
            pltpu.VMEM((8, CONV_DIM), F32),
            pltpu.VMEM((CS, DI), F32),
        ],
        compiler_params=pltpu.CompilerParams(
            dimension_semantics=("parallel", "arbitrary"),
        ),
        name="mamba2_block",
    )(x, norm_w.reshape(1, DM), wz, wxbc, wdt, convw, conv_b.reshape(1, -1),
      dt_bias.reshape(1, -1), A_log.reshape(1, -1), D.reshape(1, -1),
      gnorm_w.reshape(1, -1), out_proj_w.astype(BF))
    return out


# speedup vs baseline: 2.5164x; 1.5933x over previous
import jax
import jax.numpy as jnp
from jax import lax
from jax.experimental import pallas as pl
from jax.experimental.pallas import tpu as pltpu

B_, L_, DM = 2, 2048, 1024
DS, DC, HD = 128, 4, 64
DI = 2048
NH = DI // HD
CONV_DIM = DI + 2 * DS
CS = 256
NC = L_ // CS
EPS = 1e-5
BF = jnp.bfloat16
F32 = jnp.float32


def _silu(v):
    return v * (1.0 / (1.0 + jnp.exp(-v)))


def _softplus(v):
    return jnp.maximum(v, 0.0) + jnp.log(1.0 + jnp.exp(-jnp.abs(v)))


def _body(x_ref, nw_ref, wz_ref, wxbc_ref, wdt_ref, convw_ref, convb_ref,
          dtb_ref, alog_ref, dexp_ref, gw_ref, wout_ref, e_ref,
          out_ref, state_ref, hist_ref, y_ref):
    c = pl.program_id(1)

    @pl.when(c == 0)
    def _():
        state_ref[...] = jnp.zeros_like(state_ref)
        hist_ref[...] = jnp.zeros_like(hist_ref)

    xb = x_ref[0]
    ms = jnp.mean(xb * xb, axis=1, keepdims=True)
    xn = xb * lax.rsqrt(ms + EPS) * nw_ref[...]
    xnb = xn.astype(BF)

    dnums = (((1,), (1,)), ((), ()))
    z = lax.dot_general(xnb, wz_ref[...], dnums, preferred_element_type=F32)
    xbc_raw = lax.dot_general(xnb, wxbc_ref[...], dnums, preferred_element_type=F32)
    dtr = lax.dot_general(xnb, wdt_ref[...], dnums, preferred_element_type=F32)

    ext = jnp.concatenate([hist_ref[...], xbc_raw], axis=0)
    cw = convw_ref[...]
    conv = (ext[5:5 + CS] * cw[0:1] + ext[6:6 + CS] * cw[1:2]
            + ext[7:7 + CS] * cw[2:3] + ext[8:8 + CS] * cw[3:4]) + convb_ref[...]
    xBC = _silu(conv)
    hist_ref[5:8, :] = xbc_raw[CS - 3:CS, :]

    xs = xBC[:, :DI]
    Bm = xBC[:, DI:DI + DS]
    Cm = xBC[:, DI + DS:]

    li = lax.broadcasted_iota(jnp.int32, (CS, CS), 0)
    si = lax.broadcasted_iota(jnp.int32, (CS, CS), 1)
    mask = li >= si
    tril = jnp.where(mask, 1.0, 0.0).astype(BF)

    dt = _softplus(dtr + dtb_ref[...])
    A = -jnp.exp(alog_ref[...])
    dtA = dt * A
    dtA_hi = dtA.astype(BF)
    dtA_lo = (dtA - dtA_hi.astype(F32)).astype(BF)
    Acs = (jnp.dot(tril, dtA_hi, preferred_element_type=F32)
           + jnp.dot(tril, dtA_lo, preferred_element_type=F32))
    AcsT = jnp.swapaxes(Acs, 0, 1)
    Asum = Acs[CS - 1:CS, :]
    gam = jnp.exp(Asum)
    dec_st = jnp.exp(Asum - Acs)
    expAcs = jnp.exp(Acs)

    E = e_ref[...]
    dt_exp = jnp.dot(dt.astype(BF), E, preferred_element_type=F32)
    dec_exp = jnp.dot(dec_st.astype(BF), E, preferred_element_type=F32)
    eA_exp = jnp.dot(expAcs.astype(BF), E, preferred_element_type=F32)
    gam_exp = jnp.dot(gam.astype(BF), E, preferred_element_type=F32)

    Bmb = Bm.astype(BF)
    Cmb = Cm.astype(BF)
    CB = lax.dot_general(Cmb, Bmb, dnums, preferred_element_type=F32)
    CBmb = jnp.where(mask, CB, 0.0).astype(BF)

    Xall = xs * dt_exp
    Xb = Xall.astype(BF)
    Xw = (Xall * dec_exp).astype(BF)

    st = state_ref[...]
    Yo = jnp.dot(Cmb, st.astype(BF), preferred_element_type=F32)
    tA = (((0,), (0,)), ((), ()))
    upd = lax.dot_general(Bmb, Xw, tA, preferred_element_type=F32)
    state_ref[...] = st * gam_exp + upd

    for h0 in range(0, NH, 2):
        yds = []
        for h in (h0, h0 + 1):
            ah_col = Acs[:, h:h + 1]
            ah_row = AcsT[h:h + 1, :]
            ex = jnp.exp(jnp.minimum(ah_col - ah_row, 0.0))
            Mh = ex.astype(BF) * CBmb
            yds.append(jnp.dot(Mh, Xb[:, h * HD:(h + 1) * HD],
                               preferred_element_type=F32))
        y_ref[:, h0 * HD:(h0 + 2) * HD] = jnp.concatenate(yds, axis=1)

    y = (y_ref[...] + Yo * eA_exp + dexp_ref[...] * xs) * _silu(z)
    ms2 = jnp.mean(y * y, axis=1, keepdims=True)
    yn = y * lax.rsqrt(ms2 + EPS) * gw_ref[...]
    outp = lax.dot_general(yn.astype(BF), wout_ref[...], dnums,
                           preferred_element_type=F32)
    out_ref[0] = xb + outp


def kernel(x, norm_w, in_proj_w, conv_w, conv_b, dt_bias, A_log, D, gnorm_w,
           out_proj_w):
    wz = in_proj_w[:DI].astype(BF)
    wxbc = in_proj_w[DI:DI + CONV_DIM].astype(BF)
    wdt = in_proj_w[DI + CONV_DIM:].astype(BF)
    convw = jnp.zeros((8, CONV_DIM), F32).at[:DC].set(conv_w.T)
    dexp = jnp.repeat(D, HD).reshape(1, DI)
    emat = (jnp.arange(NH)[:, None] == (jnp.arange(DI)[None, :] // HD)
            ).astype(BF)

    grid = (B_, NC)
    const = lambda b, c: (0, 0)
    out = pl.pallas_call(
        _body,
        out_shape=jax.ShapeDtypeStruct((B_, L_, DM), F32),
        grid=grid,
        in_specs=[
            pl.BlockSpec((1, CS, DM), lambda b, c: (b, c, 0)),
            pl.BlockSpec((1, DM), const),
            pl.BlockSpec((DI, DM), const),
            pl.BlockSpec((CONV_DIM, DM), const),
            pl.BlockSpec((NH, DM), const),
            pl.BlockSpec((8, CONV_DIM), const),
            pl.BlockSpec((1, CONV_DIM), const),
            pl.BlockSpec((1, NH), const),
            pl.BlockSpec((1, NH), const),
            pl.BlockSpec((1, DI), const),
            pl.BlockSpec((1, DI), const),
            pl.BlockSpec((DM, DI), const),
            pl.BlockSpec((NH, DI), const),
        ],
        out_specs=pl.BlockSpec((1, CS, DM), lambda b, c: (b, c, 0)),
        scratch_shapes=[
            pltpu.VMEM((DS, DI), F32),
            pltpu.VMEM((8, CONV_DIM), F32),
            pltpu.VMEM((CS, DI), F32),
        ],
        compiler_params=pltpu.CompilerParams(
            dimension_semantics=("parallel", "arbitrary"),
        ),
        name="mamba2_block",
    )(x, norm_w.reshape(1, DM), wz, wxbc, wdt, convw, conv_b.reshape(1, -1),
      dt_bias.reshape(1, -1), A_log.reshape(1, -1), dexp,
      gnorm_w.reshape(1, -1), out_proj_w.astype(BF), emat)
    return out


# speedup vs baseline: 2.7333x; 1.0862x over previous
import jax
import jax.numpy as jnp
from jax import lax
from jax.experimental import pallas as pl
from jax.experimental.pallas import tpu as pltpu

B_, L_, DM = 2, 2048, 1024
DS, DC, HD = 128, 4, 64
DI = 2048
NH = DI // HD
CONV_DIM = DI + 2 * DS
DIN = 2 * DI + 2 * DS + NH
CS = 256
HS = CS // 2
NC = L_ // CS
EPS = 1e-5
BF = jnp.bfloat16
F32 = jnp.float32


def _silu(v):
    return v * (1.0 / (1.0 + jnp.exp(-v)))


def _softplus(v):
    return jnp.maximum(v, 0.0) + jnp.log(1.0 + jnp.exp(-jnp.abs(v)))


def _body(x_ref, nw_ref, win_ref, convw_ref, convb_ref,
          dtb_ref, alog_ref, d_ref, gw_ref, wout_ref, e_ref, e2_ref,
          out_ref, state_ref, hist_ref, y_ref):
    c = pl.program_id(1)

    @pl.when(c == 0)
    def _():
        state_ref[...] = jnp.zeros_like(state_ref)
        hist_ref[...] = jnp.zeros_like(hist_ref)

    xb = x_ref[0]
    ms = jnp.mean(xb * xb, axis=1, keepdims=True)
    xn = xb * lax.rsqrt(ms + EPS) * nw_ref[...]
    xnb = xn.astype(BF)

    dnums = (((1,), (1,)), ((), ()))
    zxbcdt = lax.dot_general(xnb, win_ref[...], dnums,
                             preferred_element_type=F32)
    z = zxbcdt[:, :DI]
    xbc_raw = zxbcdt[:, DI:DI + CONV_DIM]
    dtr = zxbcdt[:, DI + CONV_DIM:]

    ext = jnp.concatenate([hist_ref[...], xbc_raw], axis=0)
    cw = convw_ref[...]
    conv = (ext[5:5 + CS] * cw[0:1] + ext[6:6 + CS] * cw[1:2]
            + ext[7:7 + CS] * cw[2:3] + ext[8:8 + CS] * cw[3:4]) + convb_ref[...]
    xBC = _silu(conv)
    hist_ref[5:8, :] = xbc_raw[CS - 3:CS, :]

    xs = xBC[:, :DI]
    Bm = xBC[:, DI:DI + DS]
    Cm = xBC[:, DI + DS:]

    li = lax.broadcasted_iota(jnp.int32, (CS, CS), 0)
    si = lax.broadcasted_iota(jnp.int32, (CS, CS), 1)
    mask = li >= si
    tril = jnp.where(mask, 1.0, 0.0).astype(BF)
    ident = jnp.where(li == si, 1.0, 0.0)[:HS, :HS].astype(BF)

    dt = _softplus(dtr + dtb_ref[...])
    A = -jnp.exp(alog_ref[...])
    dtA = dt * A
    dtA_hi = dtA.astype(BF)
    dtA_lo = (dtA - dtA_hi.astype(F32)).astype(BF)
    Acs = (jnp.dot(tril, dtA_hi, preferred_element_type=F32)
           + jnp.dot(tril, dtA_lo, preferred_element_type=F32))
    AcsT = jnp.swapaxes(Acs, 0, 1)
    dtT_bf = jnp.swapaxes(dt, 0, 1).astype(BF)
    Asum = Acs[CS - 1:CS, :]
    gam = jnp.exp(Asum)
    dec_st = jnp.exp(Asum - Acs)
    expAcs = jnp.exp(Acs)

    cat = jnp.concatenate([dt * dec_st, expAcs], axis=1).astype(BF)
    big = jnp.dot(cat, e2_ref[...], preferred_element_type=F32)
    dec_exp = big[:, :DI]
    eA_exp = big[:, DI:]
    gam_exp = jnp.dot(gam.astype(BF), e_ref[...], preferred_element_type=F32)

    Bmb = Bm.astype(BF)
    Cmb = Cm.astype(BF)
    CB = lax.dot_general(Cmb, Bmb, dnums, preferred_element_type=F32)
    CBmb = jnp.where(mask, CB, 0.0).astype(BF)

    Xb = xs.astype(BF)
    Xw = (xs * dec_exp).astype(BF)

    st = state_ref[...]
    Yo = jnp.dot(Cmb, st.astype(BF), preferred_element_type=F32)
    tA = (((0,), (0,)), ((), ()))
    upd = lax.dot_general(Bmb, Xw, tA, preferred_element_type=F32)
    state_ref[...] = st * gam_exp + upd

    M11c = CBmb[:HS, :HS]
    M21c = CBmb[HS:, :HS]
    M22c = CBmb[HS:, HS:]
    for h0 in range(0, NH, 2):
        tops, bots = [], []
        for h in (h0, h0 + 1):
            a_lo = Acs[:HS, h:h + 1]
            a_hi = Acs[HS:, h:h + 1]
            r_lo = AcsT[h:h + 1, :HS]
            r_hi = AcsT[h:h + 1, HS:]
            dt_lo = dtT_bf[h:h + 1, :HS]
            dt_hi = dtT_bf[h:h + 1, HS:]
            dI = d_ref[0:1, h:h + 1].astype(BF) * ident
            M11 = (jnp.exp(jnp.minimum(a_lo - r_lo, 0.0)).astype(BF)
                   * M11c * dt_lo + dI)
            M22 = (jnp.exp(jnp.minimum(a_hi - r_hi, 0.0)).astype(BF)
                   * M22c * dt_hi + dI)
            M21 = jnp.exp(a_hi - r_lo).astype(BF) * M21c * dt_lo
            Xt = Xb[:HS, h * HD:(h + 1) * HD]
            Xo = Xb[HS:, h * HD:(h + 1) * HD]
            tops.append(jnp.dot(M11, Xt, preferred_element_type=F32))
            bots.append(jnp.dot(M21, Xt, preferred_element_type=F32)
                        + jnp.dot(M22, Xo, preferred_element_type=F32))
        y_ref[:HS, h0 * HD:(h0 + 2) * HD] = jnp.concatenate(tops, axis=1)
        y_ref[HS:, h0 * HD:(h0 + 2) * HD] = jnp.concatenate(bots, axis=1)

    y = (y_ref[...] + Yo * eA_exp) * _silu(z)
    ms2 = jnp.mean(y * y, axis=1, keepdims=True)
    yn = y * lax.rsqrt(ms2 + EPS) * gw_ref[...]
    outp = lax.dot_general(yn.astype(BF), wout_ref[...], dnums,
                           preferred_element_type=F32)
    out_ref[0] = xb + outp


def kernel(x, norm_w, in_proj_w, conv_w, conv_b, dt_bias, A_log, D, gnorm_w,
           out_proj_w):
    win = in_proj_w.astype(BF)
    convw = jnp.zeros((8, CONV_DIM), F32).at[:DC].set(conv_w.T)
    emat = (jnp.arange(NH)[:, None] == (jnp.arange(DI)[None, :] // HD)
            ).astype(BF)
    e2 = jnp.zeros((2 * NH, 2 * DI), BF)
    e2 = e2.at[:NH, :DI].set(emat).at[NH:, DI:].set(emat)

    grid = (B_, NC)
    const = lambda b, c: (0, 0)
    out = pl.pallas_call(
        _body,
        out_shape=jax.ShapeDtypeStruct((B_, L_, DM), F32),
        grid=grid,
        in_specs=[
            pl.BlockSpec((1, CS, DM), lambda b, c: (b, c, 0)),
            pl.BlockSpec((1, DM), const),
            pl.BlockSpec((DIN, DM), const),
            pl.BlockSpec((8, CONV_DIM), const),
            pl.BlockSpec((1, CONV_DIM), const),
            pl.BlockSpec((1, NH), const),
            pl.BlockSpec((1, NH), const),
            pl.BlockSpec((1, NH), const),
            pl.BlockSpec((1, DI), const),
            pl.BlockSpec((DM, DI), const),
            pl.BlockSpec((NH, DI), const),
            pl.BlockSpec((2 * NH, 2 * DI), const),
        ],
        out_specs=pl.BlockSpec((1, CS, DM), lambda b, c: (b, c, 0)),
        scratch_shapes=[
            pltpu.VMEM((DS, DI), F32),
            pltpu.VMEM((8, CONV_DIM), F32),
            pltpu.VMEM((CS, DI), F32),
        ],
        compiler_params=pltpu.CompilerParams(
            dimension_semantics=("parallel", "arbitrary"),
        ),
        name="mamba2_block",
    )(x, norm_w.reshape(1, DM), win, convw, conv_b.reshape(1, -1),
      dt_bias.reshape(1, -1), A_log.reshape(1, -1), D.reshape(1, -1),
      gnorm_w.reshape(1, -1), out_proj_w.astype(BF), emat, e2)
    return out


# speedup vs baseline: 2.8614x; 1.0469x over previous
import jax
import jax.numpy as jnp
from jax import lax
from jax.experimental import pallas as pl
from jax.experimental.pallas import tpu as pltpu

B_, L_, DM = 2, 2048, 1024
DS, DC, HD = 128, 4, 64
DI = 2048
NH = DI // HD
CONV_DIM = DI + 2 * DS
DIN = 2 * DI + 2 * DS + NH
CS = 256
HS = CS // 2
NC = L_ // CS
EPS = 1e-5
BF = jnp.bfloat16
F32 = jnp.float32


def _silu(v):
    return v * (1.0 / (1.0 + jnp.exp(-v)))


def _softplus(v):
    return jnp.maximum(v, 0.0) + jnp.log(1.0 + jnp.exp(-jnp.abs(v)))


def _body(x_ref, nw_ref, win_ref, convw_ref, convb_ref,
          dtb_ref, alog_ref, d_ref, gw_ref, wout_ref, e_ref, e2_ref,
          out_ref, state_ref, hist_ref, y_ref, winb_ref, woutb_ref):
    c = pl.program_id(1)

    @pl.when(c == 0)
    def _():
        state_ref[...] = jnp.zeros_like(state_ref)
        hist_ref[...] = jnp.zeros_like(hist_ref)
        winb_ref[...] = win_ref[...].astype(BF)
        woutb_ref[...] = wout_ref[...].astype(BF)

    xb = x_ref[0]
    ms = jnp.mean(xb * xb, axis=1, keepdims=True)
    xn = xb * lax.rsqrt(ms + EPS) * nw_ref[...]
    xnb = xn.astype(BF)

    dnums = (((1,), (1,)), ((), ()))
    zxbcdt = lax.dot_general(xnb, winb_ref[...], dnums,
                             preferred_element_type=F32)
    z = zxbcdt[:, :DI]
    xbc_raw = zxbcdt[:, DI:DI + CONV_DIM]
    dtr = zxbcdt[:, DI + CONV_DIM:]

    ext = jnp.concatenate([hist_ref[...], xbc_raw], axis=0)
    cw = convw_ref[...]
    conv = (ext[5:5 + CS] * cw[0:1] + ext[6:6 + CS] * cw[1:2]
            + ext[7:7 + CS] * cw[2:3] + ext[8:8 + CS] * cw[3:4]) + convb_ref[...]
    xBC = _silu(conv)
    hist_ref[5:8, :] = xbc_raw[CS - 3:CS, :]

    xs = xBC[:, :DI]
    Bm = xBC[:, DI:DI + DS]
    Cm = xBC[:, DI + DS:]

    li = lax.broadcasted_iota(jnp.int32, (CS, CS), 0)
    si = lax.broadcasted_iota(jnp.int32, (CS, CS), 1)
    mask = li >= si
    tril = jnp.where(mask, 1.0, 0.0).astype(BF)
    ident = jnp.where(li == si, 1.0, 0.0)[:HS, :HS].astype(BF)

    dt = _softplus(dtr + dtb_ref[...])
    A = -jnp.exp(alog_ref[...])
    dtA = dt * A
    dtA_hi = dtA.astype(BF)
    dtA_lo = (dtA - dtA_hi.astype(F32)).astype(BF)
    Acs = (jnp.dot(tril, dtA_hi, preferred_element_type=F32)
           + jnp.dot(tril, dtA_lo, preferred_element_type=F32))
    AcsT = jnp.swapaxes(Acs, 0, 1)
    dtT_bf = jnp.swapaxes(dt, 0, 1).astype(BF)
    Asum = Acs[CS - 1:CS, :]
    gam = jnp.exp(Asum)
    dec_st = jnp.exp(Asum - Acs)
    expAcs = jnp.exp(Acs)

    cat = jnp.concatenate([dt * dec_st, expAcs], axis=1).astype(BF)
    big = jnp.dot(cat, e2_ref[...], preferred_element_type=F32)
    dec_exp = big[:, :DI]
    eA_exp = big[:, DI:]
    gam_exp = jnp.dot(gam.astype(BF), e_ref[...], preferred_element_type=F32)

    Bmb = Bm.astype(BF)
    Cmb = Cm.astype(BF)
    CB = lax.dot_general(Cmb, Bmb, dnums, preferred_element_type=F32)
    CBmb = jnp.where(mask, CB, 0.0).astype(BF)

    Xb = xs.astype(BF)
    Xw = (xs * dec_exp).astype(BF)

    st = state_ref[...]
    Yo = jnp.dot(Cmb, st.astype(BF), preferred_element_type=F32)
    tA = (((0,), (0,)), ((), ()))
    upd = lax.dot_general(Bmb, Xw, tA, preferred_element_type=F32)
    state_ref[...] = st * gam_exp + upd

    M11c = CBmb[:HS, :HS]
    M21c = CBmb[HS:, :HS]
    M22c = CBmb[HS:, HS:]
    for h0 in range(0, NH, 2):
        tops, bots = [], []
        for h in (h0, h0 + 1):
            a_lo = Acs[:HS, h:h + 1]
            a_hi = Acs[HS:, h:h + 1]
            r_lo = AcsT[h:h + 1, :HS]
            r_hi = AcsT[h:h + 1, HS:]
            dt_lo = dtT_bf[h:h + 1, :HS]
            dt_hi = dtT_bf[h:h + 1, HS:]
            dI = d_ref[0:1, h:h + 1].astype(BF) * ident
            M11 = (jnp.exp(jnp.minimum(a_lo - r_lo, 0.0)).astype(BF)
                   * M11c * dt_lo + dI)
            M22 = (jnp.exp(jnp.minimum(a_hi - r_hi, 0.0)).astype(BF)
                   * M22c * dt_hi + dI)
            M21 = jnp.exp(a_hi - r_lo).astype(BF) * M21c * dt_lo
            Xt = Xb[:HS, h * HD:(h + 1) * HD]
            Xo = Xb[HS:, h * HD:(h + 1) * HD]
            tops.append(jnp.dot(M11, Xt, preferred_element_type=F32))
            bots.append(jnp.dot(M21, Xt, preferred_element_type=F32)
                        + jnp.dot(M22, Xo, preferred_element_type=F32))
        y_ref[:HS, h0 * HD:(h0 + 2) * HD] = jnp.concatenate(tops, axis=1)
        y_ref[HS:, h0 * HD:(h0 + 2) * HD] = jnp.concatenate(bots, axis=1)

    y = (y_ref[...] + Yo * eA_exp) * _silu(z)
    ms2 = jnp.mean(y * y, axis=1, keepdims=True)
    yn = y * lax.rsqrt(ms2 + EPS) * gw_ref[...]
    outp = lax.dot_general(yn.astype(BF), woutb_ref[...], dnums,
                           preferred_element_type=F32)
    out_ref[0] = xb + outp


def kernel(x, norm_w, in_proj_w, conv_w, conv_b, dt_bias, A_log, D, gnorm_w,
           out_proj_w):
    convw = jnp.zeros((8, CONV_DIM), F32).at[:DC].set(conv_w.T)
    emat = (jnp.arange(NH)[:, None] == (jnp.arange(DI)[None, :] // HD)
            ).astype(BF)
    e2 = jnp.zeros((2 * NH, 2 * DI), BF)
    e2 = e2.at[:NH, :DI].set(emat).at[NH:, DI:].set(emat)

    grid = (B_, NC)
    const = lambda b, c: (0, 0)
    out = pl.pallas_call(
        _body,
        out_shape=jax.ShapeDtypeStruct((B_, L_, DM), F32),
        grid=grid,
        in_specs=[
            pl.BlockSpec((1, CS, DM), lambda b, c: (b, c, 0)),
            pl.BlockSpec((1, DM), const),
            pl.BlockSpec((DIN, DM), const),
            pl.BlockSpec((8, CONV_DIM), const),
            pl.BlockSpec((1, CONV_DIM), const),
            pl.BlockSpec((1, NH), const),
            pl.BlockSpec((1, NH), const),
            pl.BlockSpec((1, NH), const),
            pl.BlockSpec((1, DI), const),
            pl.BlockSpec((DM, DI), const),
            pl.BlockSpec((NH, DI), const),
            pl.BlockSpec((2 * NH, 2 * DI), const),
        ],
        out_specs=pl.BlockSpec((1, CS, DM), lambda b, c: (b, c, 0)),
        scratch_shapes=[
            pltpu.VMEM((DS, DI), F32),
            pltpu.VMEM((8, CONV_DIM), F32),
            pltpu.VMEM((CS, DI), F32),
            pltpu.VMEM((DIN, DM), BF),
            pltpu.VMEM((DM, DI), BF),
        ],
        compiler_params=pltpu.CompilerParams(
            dimension_semantics=("parallel", "arbitrary"),
        ),
        name="mamba2_block",
    )(x, norm_w.reshape(1, DM), in_proj_w, convw, conv_b.reshape(1, -1),
      dt_bias.reshape(1, -1), A_log.reshape(1, -1), D.reshape(1, -1),
      gnorm_w.reshape(1, -1), out_proj_w, emat, e2)
    return out
